# baseline (speedup 1.0000x reference)
import functools

import jax
import jax.numpy as jnp
from jax import lax
from jax.experimental import pallas as pl
from jax.experimental.pallas import tpu as pltpu
from jax.experimental.pallas import tpu_sc as plsc

CODE_DIM = 32
OUT_DIM = 64
LANES = 16
DCH = OUT_DIM // LANES
MEGA = 80
RPD = 128
R = 16
UNROLL = 2


def _fire_rows(emb_hbm, codes_f, rows_v, sem, r):
    for g in range(R * CODE_DIM // RPD):
        pltpu.async_copy(emb_hbm.at[codes_f.at[r * (R * CODE_DIM // RPD) + g]],
                         rows_v.at[pl.ds(g * RPD, RPD), :], sem)


def _drain_rows(emb_hbm, codes_f, rows_v, sem):
    for g in range(R * CODE_DIM // RPD):
        pltpu.make_async_copy(emb_hbm.at[codes_f.at[g]],
                              rows_v.at[pl.ds(g * RPD, RPD), :], sem).wait()


def _pool(rows_v, out_v):
    hi_mask = jnp.full((LANES,), -65536, jnp.int32)
    iota2 = lax.iota(jnp.int32, LANES) * 2
    nb = CODE_DIM // LANES
    for k in range(R):
        def body(cc, acc):
            new = list(acc)
            for u in range(UNROLL):
                for b in range(nb):
                    w = rows_v[k * CODE_DIM + cc * UNROLL + u,
                               pl.ds(b * LANES, LANES)]
                    ev = plsc.bitcast(lax.shift_left(w, 16), jnp.float32)
                    od = plsc.bitcast(w & hi_mask, jnp.float32)
                    new[(u * nb + b) * 2] = new[(u * nb + b) * 2] + ev
                    new[(u * nb + b) * 2 + 1] = new[(u * nb + b) * 2 + 1] + od
            return tuple(new)

        acc = lax.fori_loop(
            0, CODE_DIM // UNROLL, body,
            tuple(jnp.zeros((LANES,), jnp.float32)
                  for _ in range(UNROLL * nb * 2)))
        row = jnp.full((LANES,), k, jnp.int32)
        for b in range(nb):
            ev = acc[b * 2] + acc[(nb + b) * 2]
            od = acc[b * 2 + 1] + acc[(nb + b) * 2 + 1]
            plsc.store_scatter(out_v, [row, iota2 + 2 * LANES * b], ev)
            plsc.store_scatter(out_v, [row, iota2 + 2 * LANES * b + 1], od)


def _fire_out(out_v, out_hbm, sem, start):
    pltpu.async_copy(out_v, out_hbm.at[pl.ds(start, R)], sem)


def _drain_out(out_v, out_hbm, sem, start):
    pltpu.make_async_copy(out_v, out_hbm.at[pl.ds(start, R)], sem).wait()


def _sc_body(num_workers, n_queries, item_hbm, pq_hbm, emb_hbm, out_hbm,
             ids_v, bounce_v, codes_f, rows_a, rows_b, out_a, out_b,
             sem_stage, sem_a, sem_b, sem_oa, sem_ob):
    wid = lax.axis_index("s") * 2 + lax.axis_index("c")
    qpw = n_queries // num_workers
    base = wid * qpw
    nrounds = qpw // R

    pltpu.sync_copy(item_hbm.at[pl.ds(base, qpw)], ids_v)
    frows = MEGA // 4

    lo_mask = jnp.full((LANES,), 0xFFFF, jnp.int32)

    def stage_body(m, _):
        pltpu.async_copy(
            pq_hbm.at[ids_v.at[pl.ds(m * MEGA, MEGA)]], bounce_v,
            sem_stage).wait()
        for g in range(frows):
            for j in range(4):
                w = bounce_v[g * 4 + j, pl.ds(0, LANES)]
                codes_f[m * frows + g, pl.ds(j * 2 * LANES, LANES)] = (
                    w & lo_mask)
                codes_f[m * frows + g, pl.ds((j * 2 + 1) * LANES, LANES)] = (
                    lax.shift_right_logical(w, 16))
        return 0

    lax.fori_loop(0, qpw // MEGA, stage_body, 0)

    _fire_rows(emb_hbm, codes_f, rows_a, sem_a, 0)
    _fire_rows(emb_hbm, codes_f, rows_b, sem_b, 1)

    _drain_rows(emb_hbm, codes_f, rows_a, sem_a)
    _pool(rows_a, out_a)
    _fire_out(out_a, out_hbm, sem_oa, base)
    _fire_rows(emb_hbm, codes_f, rows_a, sem_a, 2)
    _drain_rows(emb_hbm, codes_f, rows_b, sem_b)
    _pool(rows_b, out_b)
    _fire_out(out_b, out_hbm, sem_ob, base + R)
    _fire_rows(emb_hbm, codes_f, rows_b, sem_b, 3)

    def pair_body(i, _):
        r0 = 2 * i + 2
        _drain_rows(emb_hbm, codes_f, rows_a, sem_a)
        _drain_out(out_a, out_hbm, sem_oa, base)
        _pool(rows_a, out_a)
        _fire_out(out_a, out_hbm, sem_oa, base + r0 * R)
        _fire_rows(emb_hbm, codes_f, rows_a, sem_a, r0 + 2)
        _drain_rows(emb_hbm, codes_f, rows_b, sem_b)
        _drain_out(out_b, out_hbm, sem_ob, base)
        _pool(rows_b, out_b)
        _fire_out(out_b, out_hbm, sem_ob, base + (r0 + 1) * R)
        _fire_rows(emb_hbm, codes_f, rows_b, sem_b, r0 + 3)
        return 0

    lax.fori_loop(0, nrounds // 2 - 2, pair_body, 0)

    r = nrounds - 2
    _drain_rows(emb_hbm, codes_f, rows_a, sem_a)
    _drain_out(out_a, out_hbm, sem_oa, base)
    _pool(rows_a, out_a)
    _fire_out(out_a, out_hbm, sem_oa, base + r * R)
    _drain_rows(emb_hbm, codes_f, rows_b, sem_b)
    _drain_out(out_b, out_hbm, sem_ob, base)
    _pool(rows_b, out_b)
    _fire_out(out_b, out_hbm, sem_ob, base + (r + 1) * R)
    _drain_out(out_a, out_hbm, sem_oa, base)
    _drain_out(out_b, out_hbm, sem_ob, base)


def kernel(item_seq, pq_codes, emb_table):
    batch, hist = item_seq.shape
    n_queries = batch * hist
    info = plsc.get_sparse_core_info()
    num_workers = info.num_cores * info.num_subcores
    qpw = n_queries // num_workers
    assert qpw % MEGA == 0 and qpw % (2 * R) == 0 and (R * CODE_DIM) % RPD == 0

    mesh = plsc.VectorSubcoreMesh(core_axis_name="c", subcore_axis_name="s")
    run = pl.kernel(
        functools.partial(_sc_body, num_workers, n_queries),
        out_type=jax.ShapeDtypeStruct((n_queries, OUT_DIM), jnp.float32),
        mesh=mesh,
        scratch_types=[
            pltpu.VMEM((qpw,), jnp.int32),
            pltpu.VMEM((MEGA, CODE_DIM // 2), jnp.int32),
            pltpu.VMEM((qpw * CODE_DIM // RPD, RPD), jnp.int32),
            pltpu.VMEM((R * CODE_DIM, OUT_DIM // 2), jnp.int32),
            pltpu.VMEM((R * CODE_DIM, OUT_DIM // 2), jnp.int32),
            pltpu.VMEM((R, OUT_DIM), jnp.float32),
            pltpu.VMEM((R, OUT_DIM), jnp.float32),
            pltpu.SemaphoreType.DMA,
            pltpu.SemaphoreType.DMA,
            pltpu.SemaphoreType.DMA,
            pltpu.SemaphoreType.DMA,
            pltpu.SemaphoreType.DMA,
        ],
        compiler_params=pltpu.CompilerParams(use_tc_tiling_on_sc=False,
                                             needs_layout_passes=False),
    )
    emb_bf = (emb_table * (1.0 / CODE_DIM)).astype(jnp.bfloat16)
    emb_i = jax.lax.bitcast_convert_type(
        emb_bf.reshape(emb_bf.shape[0], OUT_DIM // 2, 2), jnp.int32)
    pq_pack = jax.lax.bitcast_convert_type(
        pq_codes.astype(jnp.int16).reshape(-1, CODE_DIM // 2, 2), jnp.int32)
    out = run(item_seq.reshape(n_queries), pq_pack, emb_i)
    return out.reshape(batch, hist, OUT_DIM)


# speedup vs baseline: 1.7999x; 1.7999x over previous
import functools

import jax
import jax.numpy as jnp
from jax import lax
from jax.experimental import pallas as pl
from jax.experimental.pallas import tpu as pltpu
from jax.experimental.pallas import tpu_sc as plsc

CODE_DIM = 32
OUT_DIM = 64
LANES = 16
DCH = OUT_DIM // LANES
MEGA = 80
RPD = 128
R = 16
UNROLL = 2


def _fire_rows(emb_hbm, codes_f, rows_v, sem, r):
    for g in range(R * CODE_DIM // RPD):
        pltpu.async_copy(emb_hbm.at[codes_f.at[r * (R * CODE_DIM // RPD) + g]],
                         rows_v.at[pl.ds(g * RPD, RPD), :], sem)


def _drain_rows(emb_hbm, codes_f, rows_v, sem):
    for g in range(R * CODE_DIM // RPD):
        pltpu.make_async_copy(emb_hbm.at[codes_f.at[g]],
                              rows_v.at[pl.ds(g * RPD, RPD), :], sem).wait()


def _pool(rows_v, out_v):
    hi_mask = jnp.full((LANES,), -65536, jnp.int32)
    iota2 = lax.iota(jnp.int32, LANES) * 2
    nb = CODE_DIM // LANES
    for k in range(R):
        def body(cc, acc):
            new = list(acc)
            for u in range(UNROLL):
                for b in range(nb):
                    w = rows_v[k * CODE_DIM + cc * UNROLL + u,
                               pl.ds(b * LANES, LANES)]
                    ev = plsc.bitcast(lax.shift_left(w, 16), jnp.float32)
                    od = plsc.bitcast(w & hi_mask, jnp.float32)
                    new[(u * nb + b) * 2] = new[(u * nb + b) * 2] + ev
                    new[(u * nb + b) * 2 + 1] = new[(u * nb + b) * 2 + 1] + od
            return tuple(new)

        acc = lax.fori_loop(
            0, CODE_DIM // UNROLL, body,
            tuple(jnp.zeros((LANES,), jnp.float32)
                  for _ in range(UNROLL * nb * 2)))
        row = jnp.full((LANES,), k, jnp.int32)
        for b in range(nb):
            ev = acc[b * 2] + acc[(nb + b) * 2]
            od = acc[b * 2 + 1] + acc[(nb + b) * 2 + 1]
            plsc.store_scatter(out_v, [row, iota2 + 2 * LANES * b], ev)
            plsc.store_scatter(out_v, [row, iota2 + 2 * LANES * b + 1], od)


def _fire_out(out_v, out_hbm, sem, start):
    pltpu.async_copy(out_v, out_hbm.at[pl.ds(start, R)], sem)


def _drain_out(out_v, out_hbm, sem, start):
    pltpu.make_async_copy(out_v, out_hbm.at[pl.ds(start, R)], sem).wait()


def _sc_body(num_workers, n_queries, item_hbm, pq_hbm, emb_hbm, out_hbm,
             ids_v, bounce_v, codes_f, rows_a, rows_b, out_a, out_b,
             sem_stage, sem_a, sem_b, sem_oa, sem_ob):
    wid = lax.axis_index("s") * 2 + lax.axis_index("c")
    qpw = n_queries // num_workers
    base = wid * qpw
    nrounds = qpw // R

    pltpu.sync_copy(item_hbm.at[pl.ds(base, qpw)], ids_v)
    frows = MEGA // 4

    def stage_body(m, _):
        pltpu.async_copy(
            pq_hbm.at[ids_v.at[pl.ds(m * MEGA, MEGA)]], bounce_v,
            sem_stage).wait()
        for g in range(frows):
            for j in range(8):
                codes_f[m * frows + g, pl.ds(j * LANES, LANES)] = (
                    bounce_v[g * 4 + j // 2,
                             pl.ds((j % 2) * LANES, LANES)])
        return 0

    lax.fori_loop(0, qpw // MEGA, stage_body, 0)

    _fire_rows(emb_hbm, codes_f, rows_a, sem_a, 0)
    _fire_rows(emb_hbm, codes_f, rows_b, sem_b, 1)

    _drain_rows(emb_hbm, codes_f, rows_a, sem_a)
    _pool(rows_a, out_a)
    _fire_out(out_a, out_hbm, sem_oa, base)
    _fire_rows(emb_hbm, codes_f, rows_a, sem_a, 2)
    _drain_rows(emb_hbm, codes_f, rows_b, sem_b)
    _pool(rows_b, out_b)
    _fire_out(out_b, out_hbm, sem_ob, base + R)
    _fire_rows(emb_hbm, codes_f, rows_b, sem_b, 3)

    def pair_body(i, _):
        r0 = 2 * i + 2
        _drain_rows(emb_hbm, codes_f, rows_a, sem_a)
        _drain_out(out_a, out_hbm, sem_oa, base)
        _pool(rows_a, out_a)
        _fire_out(out_a, out_hbm, sem_oa, base + r0 * R)
        _fire_rows(emb_hbm, codes_f, rows_a, sem_a, r0 + 2)
        _drain_rows(emb_hbm, codes_f, rows_b, sem_b)
        _drain_out(out_b, out_hbm, sem_ob, base)
        _pool(rows_b, out_b)
        _fire_out(out_b, out_hbm, sem_ob, base + (r0 + 1) * R)
        _fire_rows(emb_hbm, codes_f, rows_b, sem_b, r0 + 3)
        return 0

    lax.fori_loop(0, nrounds // 2 - 2, pair_body, 0)

    r = nrounds - 2
    _drain_rows(emb_hbm, codes_f, rows_a, sem_a)
    _drain_out(out_a, out_hbm, sem_oa, base)
    _pool(rows_a, out_a)
    _fire_out(out_a, out_hbm, sem_oa, base + r * R)
    _drain_rows(emb_hbm, codes_f, rows_b, sem_b)
    _drain_out(out_b, out_hbm, sem_ob, base)
    _pool(rows_b, out_b)
    _fire_out(out_b, out_hbm, sem_ob, base + (r + 1) * R)
    _drain_out(out_a, out_hbm, sem_oa, base)
    _drain_out(out_b, out_hbm, sem_ob, base)


def kernel(item_seq, pq_codes, emb_table):
    batch, hist = item_seq.shape
    n_queries = batch * hist
    info = plsc.get_sparse_core_info()
    num_workers = info.num_cores * info.num_subcores
    qpw = n_queries // num_workers
    assert qpw % MEGA == 0 and qpw % (2 * R) == 0 and (R * CODE_DIM) % RPD == 0

    mesh = plsc.VectorSubcoreMesh(core_axis_name="c", subcore_axis_name="s")
    run = pl.kernel(
        functools.partial(_sc_body, num_workers, n_queries),
        out_type=jax.ShapeDtypeStruct((n_queries, OUT_DIM), jnp.float32),
        mesh=mesh,
        scratch_types=[
            pltpu.VMEM((qpw,), jnp.int32),
            pltpu.VMEM((MEGA, CODE_DIM), jnp.int32),
            pltpu.VMEM((qpw * CODE_DIM // RPD, RPD), jnp.int32),
            pltpu.VMEM((R * CODE_DIM, OUT_DIM // 2), jnp.int32),
            pltpu.VMEM((R * CODE_DIM, OUT_DIM // 2), jnp.int32),
            pltpu.VMEM((R, OUT_DIM), jnp.float32),
            pltpu.VMEM((R, OUT_DIM), jnp.float32),
            pltpu.SemaphoreType.DMA,
            pltpu.SemaphoreType.DMA,
            pltpu.SemaphoreType.DMA,
            pltpu.SemaphoreType.DMA,
            pltpu.SemaphoreType.DMA,
        ],
        compiler_params=pltpu.CompilerParams(use_tc_tiling_on_sc=False,
                                             needs_layout_passes=False),
    )
    emb_bf = (emb_table * (1.0 / CODE_DIM)).astype(jnp.bfloat16)
    emb_i = jax.lax.bitcast_convert_type(
        emb_bf.reshape(emb_bf.shape[0], OUT_DIM // 2, 2), jnp.int32)
    out = run(item_seq.reshape(n_queries), pq_codes, emb_i)
    return out.reshape(batch, hist, OUT_DIM)
